# baseline (speedup 1.0000x reference)
import jax
import jax.numpy as jnp
from jax.experimental import pallas as pl
from jax.experimental.pallas import tpu as pltpu

_M = 8


def _moe_block(z0, z1, z2, z3, z4, z5, z6, z7, w1, b1, w2t, b2,
               fused_ref, w_ref):
    zs = [z0[:], z1[:], z2[:], z3[:], z4[:], z5[:], z6[:], z7[:]]
    dim = zs[0].shape[1]

    dn = (((0,), (1,)), ((), ()))
    parts = [jax.lax.dot_general(w1[i * dim:(i + 1) * dim, :], zs[i], dn,
                                 preferred_element_type=jnp.float32)
             for i in range(_M)]
    while len(parts) > 1:
        parts = [parts[j] + parts[j + 1] for j in range(0, len(parts), 2)]
    x = parts[0] + b1[:]
    h = 0.5 * x * (1.0 + jax.lax.erf(x * 0.7071067811865476))
    logits = jnp.dot(w2t[:], h, preferred_element_type=jnp.float32) + b2[:]

    bn = logits.shape[1]
    iota = jax.lax.broadcasted_iota(jnp.int32, (_M, bn), 0)
    m1 = jnp.max(logits, axis=0, keepdims=True)
    idx1 = jnp.min(jnp.where(logits == m1, iota, _M), axis=0, keepdims=True)
    mask1 = iota == idx1
    neg_inf = jnp.float32(-jnp.inf)
    rest = jnp.where(mask1, neg_inf, logits)
    m2 = jnp.max(rest, axis=0, keepdims=True)
    idx2 = jnp.min(jnp.where(rest == m2, iota, _M), axis=0, keepdims=True)
    mask2 = iota == idx2
    e2 = jnp.exp(m2 - m1)
    inv = 1.0 / (1.0 + e2)
    wt = jnp.where(mask1, inv, 0.0) + jnp.where(mask2, e2 * inv, 0.0)
    w = wt.T
    w_ref[:] = w

    fused = zs[0] * w[:, 0:1]
    for i in range(1, _M):
        fused = fused + zs[i] * w[:, i:i + 1]
    fused_ref[:] = fused


@jax.jit
def kernel(z0, z1, z2, z3, z4, z5, z6, z7, W1, b1, W2, b2):
    n, dim = z0.shape
    hid = W1.shape[1]
    bn = 512
    grid = (n // bn,)

    z_spec = pl.BlockSpec((bn, dim), lambda i: (i, 0))
    fused, w = pl.pallas_call(
        _moe_block,
        grid=grid,
        in_specs=[z_spec] * _M + [
            pl.BlockSpec((dim * _M, hid), lambda i: (0, 0)),
            pl.BlockSpec((hid, 1), lambda i: (0, 0)),
            pl.BlockSpec((_M, hid), lambda i: (0, 0)),
            pl.BlockSpec((_M, 1), lambda i: (0, 0)),
        ],
        out_specs=[
            pl.BlockSpec((bn, dim), lambda i: (i, 0)),
            pl.BlockSpec((bn, _M), lambda i: (i, 0)),
        ],
        out_shape=[
            jax.ShapeDtypeStruct((n, dim), jnp.float32),
            jax.ShapeDtypeStruct((n, _M), jnp.float32),
        ],
        compiler_params=pltpu.CompilerParams(
            dimension_semantics=("arbitrary",),
            vmem_limit_bytes=100 * 1024 * 1024,
        ),
    )(z0, z1, z2, z3, z4, z5, z6, z7,
      W1, b1.reshape(hid, 1), W2.T, b2.reshape(_M, 1))
    return fused, w


# speedup vs baseline: 1.0448x; 1.0448x over previous
import jax
import jax.numpy as jnp
from jax.experimental import pallas as pl
from jax.experimental.pallas import tpu as pltpu

_M = 8


def _moe_block(z0, z1, z2, z3, z4, z5, z6, z7, w1t, b1, w2t, b2,
               fused_ref, w_ref):
    zs = [z0[:], z1[:], z2[:], z3[:], z4[:], z5[:], z6[:], z7[:]]
    dim = zs[0].shape[1]

    dn = (((1,), (1,)), ((), ()))
    parts = [jax.lax.dot_general(w1t[:, i * dim:(i + 1) * dim], zs[i], dn,
                                 preferred_element_type=jnp.float32)
             for i in range(_M)]
    while len(parts) > 1:
        parts = [parts[j] + parts[j + 1] for j in range(0, len(parts), 2)]
    x = parts[0] + b1[:]
    h = 0.5 * x * (1.0 + jax.lax.erf(x * 0.7071067811865476))
    logits = jnp.dot(w2t[:], h, preferred_element_type=jnp.float32) + b2[:]

    bn = logits.shape[1]
    iota = jax.lax.broadcasted_iota(jnp.int32, (_M, bn), 0)
    m1 = jnp.max(logits, axis=0, keepdims=True)
    idx1 = jnp.min(jnp.where(logits == m1, iota, _M), axis=0, keepdims=True)
    mask1 = iota == idx1
    neg_inf = jnp.float32(-jnp.inf)
    rest = jnp.where(mask1, neg_inf, logits)
    m2 = jnp.max(rest, axis=0, keepdims=True)
    idx2 = jnp.min(jnp.where(rest == m2, iota, _M), axis=0, keepdims=True)
    mask2 = iota == idx2
    e2 = jnp.exp(m2 - m1)
    inv = 1.0 / (1.0 + e2)
    wt = jnp.where(mask1, inv, 0.0) + jnp.where(mask2, e2 * inv, 0.0)
    w = wt.T
    w_ref[:] = w

    fused = zs[0] * w[:, 0:1]
    for i in range(1, _M):
        fused = fused + zs[i] * w[:, i:i + 1]
    fused_ref[:] = fused


@jax.jit
def kernel(z0, z1, z2, z3, z4, z5, z6, z7, W1, b1, W2, b2):
    n, dim = z0.shape
    hid = W1.shape[1]
    bn = 512
    grid = (n // bn,)

    z_spec = pl.BlockSpec((bn, dim), lambda i: (i, 0))
    fused, w = pl.pallas_call(
        _moe_block,
        grid=grid,
        in_specs=[z_spec] * _M + [
            pl.BlockSpec((hid, dim * _M), lambda i: (0, 0)),
            pl.BlockSpec((hid, 1), lambda i: (0, 0)),
            pl.BlockSpec((_M, hid), lambda i: (0, 0)),
            pl.BlockSpec((_M, 1), lambda i: (0, 0)),
        ],
        out_specs=[
            pl.BlockSpec((bn, dim), lambda i: (i, 0)),
            pl.BlockSpec((bn, _M), lambda i: (i, 0)),
        ],
        out_shape=[
            jax.ShapeDtypeStruct((n, dim), jnp.float32),
            jax.ShapeDtypeStruct((n, _M), jnp.float32),
        ],
        compiler_params=pltpu.CompilerParams(
            dimension_semantics=("arbitrary",),
            vmem_limit_bytes=100 * 1024 * 1024,
        ),
    )(z0, z1, z2, z3, z4, z5, z6, z7,
      W1.T, b1.reshape(hid, 1), W2.T, b2.reshape(_M, 1))
    return fused, w
